# baseline (speedup 1.0000x reference)
import functools

import jax
import jax.numpy as jnp
from jax import lax
from jax.experimental import pallas as pl
from jax.experimental.pallas import tpu as pltpu
from jax.experimental.pallas import tpu_sc as plsc

N_VOCAB = 1000000
N_EMBED = 64
BATCH = 16384
N_SAMPLES = 5

NC = 2
NS = 16
NW = NC * NS
CHUNK = 128

IN_CH = BATCH // (NW * CHUNK)
NZ_CH = BATCH * N_SAMPLES // (NW * CHUNK)

_mesh = plsc.VectorSubcoreMesh(core_axis_name="c", subcore_axis_name="s")


@functools.partial(
    pl.kernel,
    mesh=_mesh,
    compiler_params=pltpu.CompilerParams(use_tc_tiling_on_sc=False),
    out_type=[
        jax.ShapeDtypeStruct((BATCH, N_EMBED), jnp.float32),
        jax.ShapeDtypeStruct((BATCH, N_EMBED), jnp.float32),
        jax.ShapeDtypeStruct((BATCH * N_SAMPLES, N_EMBED), jnp.float32),
    ],
    scratch_types=[
        pltpu.VMEM((IN_CH, CHUNK), jnp.int32),
        pltpu.VMEM((IN_CH, CHUNK), jnp.int32),
        pltpu.VMEM((NZ_CH, CHUNK), jnp.int32),
        pltpu.VMEM((CHUNK, N_EMBED), jnp.float32),
        pltpu.SemaphoreType.DMA,
    ],
)
def _gather3(in_tab, out_tab, idx_in, idx_out, idx_nz,
             o_in, o_out, o_nz, iv_in, iv_out, iv_nz, rows, sem):
    w = lax.axis_index("s") * NC + lax.axis_index("c")
    pltpu.sync_copy(idx_in.at[w], iv_in)
    pltpu.sync_copy(idx_out.at[w], iv_out)
    pltpu.sync_copy(idx_nz.at[w], iv_nz)
    for j in range(IN_CH):
        pltpu.async_copy(in_tab.at[iv_in.at[j]], rows, sem).wait()
        pltpu.sync_copy(rows, o_in.at[pl.ds(w * IN_CH * CHUNK + j * CHUNK, CHUNK)])
    for j in range(IN_CH):
        pltpu.async_copy(out_tab.at[iv_out.at[j]], rows, sem).wait()
        pltpu.sync_copy(rows, o_out.at[pl.ds(w * IN_CH * CHUNK + j * CHUNK, CHUNK)])
    for j in range(NZ_CH):
        pltpu.async_copy(out_tab.at[iv_nz.at[j]], rows, sem).wait()
        pltpu.sync_copy(rows, o_nz.at[pl.ds(w * NZ_CH * CHUNK + j * CHUNK, CHUNK)])


def kernel(in_embed_weight, out_embed_weight, input_words, output_words, noise_words):
    idx_in = input_words.astype(jnp.int32).reshape(NW, IN_CH, CHUNK)
    idx_out = output_words.astype(jnp.int32).reshape(NW, IN_CH, CHUNK)
    idx_nz = noise_words.astype(jnp.int32).reshape(NW, NZ_CH, CHUNK)
    o_in, o_out, o_nz = _gather3(
        in_embed_weight, out_embed_weight, idx_in, idx_out, idx_nz)
    return (o_in, o_out, o_nz.reshape(BATCH, N_SAMPLES, N_EMBED))


# speedup vs baseline: 1.0148x; 1.0148x over previous
import functools

import jax
import jax.numpy as jnp
from jax import lax
from jax.experimental import pallas as pl
from jax.experimental.pallas import tpu as pltpu
from jax.experimental.pallas import tpu_sc as plsc

N_VOCAB = 1000000
N_EMBED = 64
BATCH = 16384
N_SAMPLES = 5

NC = 2
NS = 16
NW = NC * NS
CHUNK = 128

IN_CH = BATCH // (NW * CHUNK)
NZ_CH = BATCH * N_SAMPLES // (NW * CHUNK)

_mesh = plsc.VectorSubcoreMesh(core_axis_name="c", subcore_axis_name="s")


@functools.partial(
    pl.kernel,
    mesh=_mesh,
    compiler_params=pltpu.CompilerParams(use_tc_tiling_on_sc=False),
    out_type=[
        jax.ShapeDtypeStruct((BATCH, N_EMBED), jnp.float32),
        jax.ShapeDtypeStruct((BATCH, N_EMBED), jnp.float32),
        jax.ShapeDtypeStruct((BATCH * N_SAMPLES, N_EMBED), jnp.float32),
    ],
    scratch_types=[
        pltpu.VMEM((IN_CH, CHUNK), jnp.int32),
        pltpu.VMEM((IN_CH, CHUNK), jnp.int32),
        pltpu.VMEM((NZ_CH, CHUNK), jnp.int32),
        pltpu.VMEM((8, CHUNK, N_EMBED), jnp.float32),
        pltpu.SemaphoreType.DMA,
        pltpu.SemaphoreType.DMA,
        pltpu.SemaphoreType.DMA,
        pltpu.SemaphoreType.DMA,
        pltpu.SemaphoreType.DMA,
        pltpu.SemaphoreType.DMA,
        pltpu.SemaphoreType.DMA,
        pltpu.SemaphoreType.DMA,
        pltpu.SemaphoreType.DMA,
        pltpu.SemaphoreType.DMA,
        pltpu.SemaphoreType.DMA,
        pltpu.SemaphoreType.DMA,
        pltpu.SemaphoreType.DMA,
        pltpu.SemaphoreType.DMA,
        pltpu.SemaphoreType.DMA,
        pltpu.SemaphoreType.DMA,
    ],
)
def _gather3(in_tab, out_tab, idx_in, idx_out, idx_nz,
             o_in, o_out, o_nz, iv_in, iv_out, iv_nz, bufs, *sems):
    NBUF = 8
    gsem = sems[:NBUF]
    ssem = sems[NBUF:]
    w = lax.axis_index("s") * NC + lax.axis_index("c")
    pltpu.sync_copy(idx_in.at[w], iv_in)
    pltpu.sync_copy(idx_out.at[w], iv_out)
    pltpu.sync_copy(idx_nz.at[w], iv_nz)
    chunks = (
        [(in_tab, iv_in.at[j], o_in, w * IN_CH * CHUNK + j * CHUNK)
         for j in range(IN_CH)]
        + [(out_tab, iv_out.at[j], o_out, w * IN_CH * CHUNK + j * CHUNK)
           for j in range(IN_CH)]
        + [(out_tab, iv_nz.at[j], o_nz, w * NZ_CH * CHUNK + j * CHUNK)
           for j in range(NZ_CH)]
    )
    total = len(chunks)
    gdesc = [None] * NBUF
    sdesc = [None] * NBUF
    for j in range(total):
        slot = j % NBUF
        if j >= NBUF:
            sdesc[slot].wait()
        tab, idx, out, base = chunks[j]
        gdesc[slot] = pltpu.make_async_copy(tab.at[idx], bufs.at[slot], gsem[slot])
        gdesc[slot].start()
        if j >= 1:
            p = (j - 1) % NBUF
            gdesc[p].wait()
            _, _, pout, pbase = chunks[j - 1]
            sdesc[p] = pltpu.make_async_copy(
                bufs.at[p], pout.at[pl.ds(pbase, CHUNK)], ssem[p])
            sdesc[p].start()
    last = (total - 1) % NBUF
    gdesc[last].wait()
    tab, idx, out, base = chunks[total - 1]
    sdesc[last] = pltpu.make_async_copy(bufs.at[last], out.at[pl.ds(base, CHUNK)],
                                        ssem[last])
    sdesc[last].start()
    for p in range(NBUF):
        if sdesc[p] is not None:
            sdesc[p].wait()


def kernel(in_embed_weight, out_embed_weight, input_words, output_words, noise_words):
    idx_in = input_words.astype(jnp.int32).reshape(NW, IN_CH, CHUNK)
    idx_out = output_words.astype(jnp.int32).reshape(NW, IN_CH, CHUNK)
    idx_nz = noise_words.astype(jnp.int32).reshape(NW, NZ_CH, CHUNK)
    o_in, o_out, o_nz = _gather3(
        in_embed_weight, out_embed_weight, idx_in, idx_out, idx_nz)
    return (o_in, o_out, o_nz.reshape(BATCH, N_SAMPLES, N_EMBED))


# speedup vs baseline: 2.1311x; 2.0999x over previous
import functools

import jax
import jax.numpy as jnp
from jax import lax
from jax.experimental import pallas as pl
from jax.experimental.pallas import tpu as pltpu
from jax.experimental.pallas import tpu_sc as plsc

N_VOCAB = 1000000
N_EMBED = 64
BATCH = 16384
N_SAMPLES = 5

NC = 2
NS = 16
NW = NC * NS
BLK = 8
CHUNK = 128
NBUF = 4

IN_CH = BATCH // (NW * CHUNK)
NZ_CH = BATCH * N_SAMPLES // (NW * CHUNK)

_mesh = plsc.VectorSubcoreMesh(core_axis_name="c", subcore_axis_name="s")


@functools.partial(
    pl.kernel,
    mesh=_mesh,
    compiler_params=pltpu.CompilerParams(needs_layout_passes=False),
    out_type=[
        jax.ShapeDtypeStruct((BATCH, N_EMBED), jnp.float32),
        jax.ShapeDtypeStruct((BATCH, N_EMBED), jnp.float32),
        jax.ShapeDtypeStruct((BATCH * N_SAMPLES, N_EMBED), jnp.float32),
    ],
    scratch_types=[
        pltpu.VMEM((IN_CH, CHUNK), jnp.int32),
        pltpu.VMEM((IN_CH, CHUNK), jnp.int32),
        pltpu.VMEM((IN_CH, CHUNK), jnp.int32),
        pltpu.VMEM((IN_CH, CHUNK), jnp.int32),
        pltpu.VMEM((NZ_CH, CHUNK), jnp.int32),
        pltpu.VMEM((NZ_CH, CHUNK), jnp.int32),
        pltpu.VMEM((NBUF, CHUNK, N_EMBED), jnp.float32),
        pltpu.SemaphoreType.DMA,
        pltpu.SemaphoreType.DMA,
        pltpu.SemaphoreType.DMA,
        pltpu.SemaphoreType.DMA,
        pltpu.SemaphoreType.DMA,
        pltpu.SemaphoreType.DMA,
        pltpu.SemaphoreType.DMA,
        pltpu.SemaphoreType.DMA,
    ],
)
def _gather3(in_tab, out_tab, blk_in, rem_in, blk_out, rem_out, blk_nz, rem_nz,
             o_in, o_out, o_nz,
             bi, ri, bo, ro, bn, rn, bufs, *sems):
    gsem = sems[:NBUF]
    ssem = sems[NBUF:]
    w = lax.axis_index("s") * NC + lax.axis_index("c")
    pltpu.sync_copy(blk_in.at[w], bi)
    pltpu.sync_copy(rem_in.at[w], ri)
    pltpu.sync_copy(blk_out.at[w], bo)
    pltpu.sync_copy(rem_out.at[w], ro)
    pltpu.sync_copy(blk_nz.at[w], bn)
    pltpu.sync_copy(rem_nz.at[w], rn)

    def run_task(tab, blks, rems, out, nch, wbase):
        def issue_rows(slot, j):
            def group(g, carry):
                bv = blks[j, pl.ds(g * 16, 16)]
                rv = rems[j, pl.ds(g * 16, 16)]
                for m in range(16):
                    pltpu.async_copy(tab.at[bv[m], rv[m]],
                                     bufs.at[slot, g * 16 + m], gsem[slot])
                return carry
            lax.fori_loop(0, CHUNK // 16, group, 0)

        def drain_rows(slot, j):
            pltpu.make_async_copy(
                out.at[pl.ds(wbase + j * CHUNK, CHUNK)], bufs.at[slot],
                gsem[slot]).wait()

        def s_desc(slot, j):
            return pltpu.make_async_copy(
                bufs.at[slot], out.at[pl.ds(wbase + j * CHUNK, CHUNK)],
                ssem[slot])

        for b in range(2):
            issue_rows(b, b)

        def body(i, carry):
            for b in range(NBUF):
                j = i * NBUF + b

                @pl.when(j - 2 >= 0)
                def _():
                    s_desc((b + 2) % NBUF, j - 2).wait()

                @pl.when(j + 2 < nch)
                def _():
                    issue_rows((b + 2) % NBUF, j + 2)

                drain_rows(b, j)
                s_desc(b, j).start()
            return carry

        lax.fori_loop(0, nch // NBUF, body, 0)
        s_desc((nch - 2) % NBUF, nch - 2).wait()
        s_desc((nch - 1) % NBUF, nch - 1).wait()

    run_task(in_tab, bi, ri, o_in, IN_CH, w * IN_CH * CHUNK)
    run_task(out_tab, bo, ro, o_out, IN_CH, w * IN_CH * CHUNK)
    run_task(out_tab, bn, rn, o_nz, NZ_CH, w * NZ_CH * CHUNK)


def kernel(in_embed_weight, out_embed_weight, input_words, output_words, noise_words):
    tab_in = in_embed_weight.reshape(N_VOCAB // BLK, BLK, N_EMBED)
    tab_out = out_embed_weight.reshape(N_VOCAB // BLK, BLK, N_EMBED)

    def split(words, nch):
        wi = words.astype(jnp.int32)
        blk = (wi >> 3).reshape(NW, nch, CHUNK)
        rem = (wi & 7).reshape(NW, nch, CHUNK)
        return blk, rem

    blk_in, rem_in = split(input_words, IN_CH)
    blk_out, rem_out = split(output_words, IN_CH)
    blk_nz, rem_nz = split(noise_words, NZ_CH)
    o_in, o_out, o_nz = _gather3(
        tab_in, tab_out, blk_in, rem_in, blk_out, rem_out, blk_nz, rem_nz)
    return (o_in, o_out, o_nz.reshape(BATCH, N_SAMPLES, N_EMBED))
